# Initial kernel scaffold; baseline (speedup 1.0000x reference)
import jax
import jax.numpy as jnp
from jax.experimental import pallas as pl


def kernel(x, W, b):
    raise NotImplementedError("write your pallas kernel here")


# speedup vs baseline: 2.1350x; 2.1350x over previous
import jax
import jax.numpy as jnp
from jax import lax
from jax.experimental import pallas as pl
from jax.experimental.pallas import tpu as pltpu
from jax.experimental.pallas import tpu_sc as plsc

DIM = 768
MDIM = DIM // 4
N = 8192
NH = N // 2
R = 2048
NKEEP = N - R
B = 2

BLK1 = 512
BM = 512
CW = 128
NCH = DIM // CW
NC, NS = 2, 16
CPC = NCH // NC
PT = N // NS
QT = NKEEP // NS

MININT = -(2 ** 31)


def _mnorm_body(xr_ref, w_ref, b_ref, a_ref, bb_ref):
    xab = xr_ref[...]
    w = w_ref[...]
    bv = b_ref[...]
    ma = lax.dot_general(xab[:, :DIM], w, (((1,), (1,)), ((), ())),
                         preferred_element_type=jnp.float32) + bv
    mb = lax.dot_general(xab[:, DIM:], w, (((1,), (1,)), ((), ())),
                         preferred_element_type=jnp.float32) + bv
    a_ref[...] = ma / jnp.sqrt(jnp.sum(ma * ma, axis=1, keepdims=True))
    bb_ref[...] = mb / jnp.sqrt(jnp.sum(mb * mb, axis=1, keepdims=True))


def _mnorm1(xb, W, bvec):
    xr = xb.reshape(NH, 2 * DIM)
    return pl.pallas_call(
        _mnorm_body,
        grid=(NH // BLK1,),
        in_specs=[
            pl.BlockSpec((BLK1, 2 * DIM), lambda j: (j, 0)),
            pl.BlockSpec((MDIM, DIM), lambda j: (0, 0)),
            pl.BlockSpec((1, MDIM), lambda j: (0, 0)),
        ],
        out_specs=[
            pl.BlockSpec((BLK1, MDIM), lambda j: (j, 0)),
            pl.BlockSpec((BLK1, MDIM), lambda j: (j, 0)),
        ],
        out_shape=[
            jax.ShapeDtypeStruct((NH, MDIM), jnp.float32),
            jax.ShapeDtypeStruct((NH, MDIM), jnp.float32),
        ],
    )(xr, W, bvec.reshape(1, MDIM))


def _scores_body(a_ref, bb_ref, val_ref, idx_ref):
    a = a_ref[...]
    bt = bb_ref[...]
    s = lax.dot_general(a, bt, (((1,), (1,)), ((), ())),
                        preferred_element_type=jnp.float32)
    val_ref[0, 0] = jnp.max(s, axis=1)
    idx_ref[0, 0] = jnp.argmax(s, axis=1).astype(jnp.int32)


def _scores1(a, bb):
    nblk = NH // BM
    vals, nidx = pl.pallas_call(
        _scores_body,
        grid=(nblk,),
        in_specs=[
            pl.BlockSpec((BM, MDIM), lambda j: (j, 0)),
            pl.BlockSpec((NH, MDIM), lambda j: (0, 0)),
        ],
        out_specs=[
            pl.BlockSpec((1, 1, BM), lambda j: (j, 0, 0)),
            pl.BlockSpec((1, 1, BM), lambda j: (j, 0, 0)),
        ],
        out_shape=[
            jax.ShapeDtypeStruct((nblk, 1, BM), jnp.float32),
            jax.ShapeDtypeStruct((nblk, 1, BM), jnp.int32),
        ],
    )(a, bb)
    return vals.reshape(NH), nidx.reshape(NH)


def _cumsum_rows(x):
    li = lax.broadcasted_iota(jnp.int32, (128, 128), 0)
    lj = lax.broadcasted_iota(jnp.int32, (128, 128), 1)
    tri = (li <= lj).astype(jnp.float32)
    intra = lax.dot_general(x, tri, (((1,), (0,)), ((), ())),
                            preferred_element_type=jnp.float32)
    rowsum = jnp.sum(x, axis=1)
    si = lax.broadcasted_iota(jnp.int32, (32, 32), 0)
    sj = lax.broadcasted_iota(jnp.int32, (32, 32), 1)
    stri = (si < sj).astype(jnp.float32)
    off = lax.dot_general(rowsum, stri, (((0,), (0,)), ((), ())),
                          preferred_element_type=jnp.float32)
    return intra + off[:, None]


def _select_body(v_ref, sel_ref, ne_ref, no_ref):
    v = v_ref[...]
    bits = lax.bitcast_convert_type(v, jnp.int32)
    ikey = jnp.where(bits >= 0, bits, bits ^ jnp.int32(0x7FFFFFFF))
    mi = jnp.int32(MININT)
    tu = jnp.zeros((1, 1), jnp.int32)
    for bit in range(31, -1, -1):
        m = mi if bit == 31 else jnp.int32(1 << bit)
        cand_u = tu | m
        cand_s = cand_u ^ mi
        c = jnp.sum((ikey >= cand_s).astype(jnp.int32), axis=(0, 1),
                    keepdims=True)
        tu = jnp.where(c >= R, cand_u, tu)
    thr = tu ^ mi
    gt = ikey > thr
    eq = ikey == thr
    need = jnp.int32(R) - jnp.sum(gt.astype(jnp.int32), axis=(0, 1),
                                  keepdims=True)
    eqf = eq.astype(jnp.float32)
    r_excl = _cumsum_rows(eqf) - eqf
    sel = gt | (eq & (r_excl < need.astype(jnp.float32)))
    sel32 = sel.astype(jnp.int32)
    s_incl = _cumsum_rows(sel.astype(jnp.float32)).astype(jnp.int32)
    sub = lax.broadcasted_iota(jnp.int32, (32, 128), 0)
    lane = lax.broadcasted_iota(jnp.int32, (32, 128), 1)
    k = sub * 128 + lane
    sel_ref[...] = sel32
    ne_ref[...] = 2 * k - (s_incl - sel32)
    no_ref[...] = 2 * k + 1 - s_incl


def _select1(values):
    vr = values.reshape(32, 128)
    sel, ne, no = pl.pallas_call(
        _select_body,
        out_shape=[jax.ShapeDtypeStruct((32, 128), jnp.int32)] * 3,
    )(vr)
    return sel.reshape(NH), ne.reshape(NH), no.reshape(NH)


def _merge1(xb, sel, ne, no, nidx):
    mesh = plsc.VectorSubcoreMesh(core_axis_name="c", subcore_axis_name="s",
                                  num_cores=NC, num_subcores=NS)

    def body(x_hbm, sel_hbm, ne_hbm, no_hbm, nidx_hbm, xf_hbm, own_hbm,
             cnt_hbm, sel_t, ne_t, no_t, nidx_t, slots2d, cnt_t, xbuf0,
             xbuf1, zbuf, sem_l, sem_a, accum_sp):
        core = lax.axis_index("c")
        t = lax.axis_index("s")
        zero16 = jnp.zeros((16,), jnp.float32)
        ones16 = jnp.ones((16,), jnp.float32)
        iota16 = lax.iota(jnp.int32, 16)

        pltpu.sync_copy(sel_hbm, sel_t)
        pltpu.sync_copy(ne_hbm, ne_t)
        pltpu.sync_copy(no_hbm, no_t)
        pltpu.sync_copy(nidx_hbm, nidx_t)

        def _zero_cnt(m, carry):
            cnt_t[pl.ds(m * 16, 16)] = zero16
            return carry
        lax.fori_loop(0, NKEEP // 16, _zero_cnt, 0)

        def _zero_zbuf(m, carry):
            zbuf[m // 8, pl.ds((m % 8) * 16, 16)] = zero16
            return carry
        lax.fori_loop(0, 64 * 8, _zero_zbuf, 0)

        def _slots(j, carry):
            i16 = j * 16 + iota16
            selv = sel_t[pl.ds(j * 16, 16)]
            nov = no_t[pl.ds(j * 16, 16)]
            niv = nidx_t[pl.ds(j * 16, 16)]
            nev = ne_t[pl.ds(j * 16, 16)]
            tgt = plsc.load_gather(ne_t, [niv])
            ownodd = jnp.where(selv > 0, tgt, nov)
            pe = i16 * 2
            po = pe + 1
            plsc.store_scatter(slots2d, [pe >> 7, pe & 127], nev)
            plsc.store_scatter(slots2d, [po >> 7, po & 127], ownodd)
            plsc.addupdate_scatter(cnt_t, [nev], ones16)
            plsc.addupdate_scatter(cnt_t, [ownodd], ones16)
            return carry
        lax.fori_loop(0, NH // 16, _slots, 0)

        @pl.when(core == 0)
        def _():
            pltpu.sync_copy(slots2d.at[pl.ds(t * 4, 4)],
                            own_hbm.at[pl.ds(t * 4, 4)])
            pltpu.sync_copy(cnt_t.at[pl.ds(t * QT, QT)],
                            cnt_hbm.at[pl.ds(t * QT, QT)])

        def xsrc(g):
            cl, u = g // 4, g % 4
            col = (core * CPC + cl) * CW
            return x_hbm.at[pl.ds(t * PT + u * 128, 128), pl.ds(col, CW)]

        nblk = 4 * CPC
        for z in range(QT // 64):
            pltpu.sync_copy(zbuf, accum_sp.at[pl.ds(t * QT + z * 64, 64)])
        lds = [None] * (nblk + 2)
        ads = [None] * nblk
        lds[0] = pltpu.async_copy(xsrc(0), xbuf0, sem_l)
        lds[1] = pltpu.async_copy(xsrc(1), xbuf1, sem_l)
        plsc.subcore_barrier()
        for cl in range(CPC):
            for u in range(4):
                g = 4 * cl + u
                lds[g].wait()
                buf = xbuf0 if g % 2 == 0 else xbuf1
                ads[g] = pltpu.async_copy(buf,
                                          accum_sp.at[slots2d.at[t * 4 + u]],
                                          sem_a, add=True)
                ads[g].wait()
                if g + 2 < nblk:
                    lds[g + 2] = pltpu.async_copy(xsrc(g + 2), buf, sem_l)
            plsc.subcore_barrier()
            col = (core * CPC + cl) * CW
            pltpu.sync_copy(accum_sp.at[pl.ds(t * QT, QT)],
                            xf_hbm.at[pl.ds(t * QT, QT), pl.ds(col, CW)])
            if cl + 1 < CPC:
                for z in range(QT // 64):
                    pltpu.sync_copy(zbuf,
                                    accum_sp.at[pl.ds(t * QT + z * 64, 64)])
            plsc.subcore_barrier()

    run = pl.kernel(
        body,
        out_type=(
            jax.ShapeDtypeStruct((NKEEP, DIM), jnp.float32),
            jax.ShapeDtypeStruct((64, 128), jnp.int32),
            jax.ShapeDtypeStruct((NKEEP,), jnp.float32),
        ),
        mesh=mesh,
        compiler_params=pltpu.CompilerParams(needs_layout_passes=False),
        scratch_types=[
            pltpu.VMEM((NH,), jnp.int32),
            pltpu.VMEM((NH,), jnp.int32),
            pltpu.VMEM((NH,), jnp.int32),
            pltpu.VMEM((NH,), jnp.int32),
            pltpu.VMEM((64, 128), jnp.int32),
            pltpu.VMEM((NKEEP,), jnp.float32),
            pltpu.VMEM((128, CW), jnp.float32),
            pltpu.VMEM((128, CW), jnp.float32),
            pltpu.VMEM((64, 128), jnp.float32),
            pltpu.SemaphoreType.DMA,
            pltpu.SemaphoreType.DMA,
            pltpu.VMEM_SHARED((NKEEP, CW), jnp.float32),
        ],
    )
    xf_raw, own, cnt = run(xb, sel, ne, no, nidx)
    return xf_raw, own.reshape(N), cnt


def _divide_body(x0_ref, c0_ref, x1_ref, c1_ref, out_ref):
    i = pl.program_id(0)

    @pl.when(i == 0)
    def _():
        out_ref[0] = x0_ref[...] / c0_ref[...]

    @pl.when(i == 1)
    def _():
        out_ref[0] = x1_ref[...] / c1_ref[...]


def _divide2(xr0, cnt0, xr1, cnt1):
    blk = 512

    def pin0(i, j):
        return (jnp.where(i == 0, j, 0), 0)

    def pin1(i, j):
        return (jnp.where(i == 1, j, 0), 0)

    return pl.pallas_call(
        _divide_body,
        grid=(B, NKEEP // blk),
        in_specs=[
            pl.BlockSpec((blk, DIM), pin0),
            pl.BlockSpec((blk, 1), pin0),
            pl.BlockSpec((blk, DIM), pin1),
            pl.BlockSpec((blk, 1), pin1),
        ],
        out_specs=pl.BlockSpec((1, blk, DIM), lambda i, j: (i, j, 0)),
        out_shape=jax.ShapeDtypeStruct((B, NKEEP, DIM), jnp.float32),
    )(xr0, cnt0.reshape(NKEEP, 1), xr1, cnt1.reshape(NKEEP, 1))


def kernel(x, W, b):
    per_batch = []
    for bi in range(B):
        xb = x[bi]
        a, bb = _mnorm1(xb, W, b)
        values, nidx = _scores1(a, bb)
        sel, ne, no = _select1(values)
        xfr, own, cnt = _merge1(xb, sel, ne, no, nidx)
        per_batch.append((xfr, own, cnt))
    (xr0, own0, cnt0), (xr1, own1, cnt1) = per_batch
    xf = _divide2(xr0, cnt0, xr1, cnt1)
    own = jnp.stack([own0, own1])
    return (xf, own)


# speedup vs baseline: 2.4296x; 1.1380x over previous
import functools

import jax
import jax.numpy as jnp
from jax import lax
from jax.experimental import pallas as pl
from jax.experimental.pallas import tpu as pltpu
from jax.experimental.pallas import tpu_sc as plsc

DIM = 768
MDIM = DIM // 4
N = 8192
NH = N // 2
R = 2048
NKEEP = N - R
B = 2

BLK1 = 512
BM = 512
CW = 128
NCH = DIM // CW
NC, NS = 2, 16
PT = N // NS
QT = NKEEP // NS

MININT = -(2 ** 31)


def _mnorm_body(xr_ref, w_ref, b_ref, a_ref, bb_ref):
    xab = xr_ref[0]
    w = w_ref[...]
    bv = b_ref[...]
    ma = lax.dot_general(xab[:, :DIM], w, (((1,), (1,)), ((), ())),
                         preferred_element_type=jnp.float32) + bv
    mb = lax.dot_general(xab[:, DIM:], w, (((1,), (1,)), ((), ())),
                         preferred_element_type=jnp.float32) + bv
    a_ref[0] = ma / jnp.sqrt(jnp.sum(ma * ma, axis=1, keepdims=True))
    bb_ref[0] = mb / jnp.sqrt(jnp.sum(mb * mb, axis=1, keepdims=True))


def _mnorm(x, W, b):
    xr = x.reshape(B, NH, 2 * DIM)
    return pl.pallas_call(
        _mnorm_body,
        grid=(B, NH // BLK1),
        in_specs=[
            pl.BlockSpec((1, BLK1, 2 * DIM), lambda i, j: (i, j, 0)),
            pl.BlockSpec((MDIM, DIM), lambda i, j: (0, 0)),
            pl.BlockSpec((1, MDIM), lambda i, j: (0, 0)),
        ],
        out_specs=[
            pl.BlockSpec((1, BLK1, MDIM), lambda i, j: (i, j, 0)),
            pl.BlockSpec((1, BLK1, MDIM), lambda i, j: (i, j, 0)),
        ],
        out_shape=[
            jax.ShapeDtypeStruct((B, NH, MDIM), jnp.float32),
            jax.ShapeDtypeStruct((B, NH, MDIM), jnp.float32),
        ],
    )(xr, W, b.reshape(1, MDIM))


NBLK2 = NH // BM


def _cumsum_rows(x):
    li = lax.broadcasted_iota(jnp.int32, (BM, BM), 0)
    lj = lax.broadcasted_iota(jnp.int32, (BM, BM), 1)
    tri = (li <= lj).astype(jnp.float32)
    intra = lax.dot_general(x, tri, (((1,), (0,)), ((), ())),
                            preferred_element_type=jnp.float32)
    rowsum = jnp.sum(x, axis=1)
    si = lax.broadcasted_iota(jnp.int32, (NBLK2, NBLK2), 0)
    sj = lax.broadcasted_iota(jnp.int32, (NBLK2, NBLK2), 1)
    stri = (si < sj).astype(jnp.float32)
    off = lax.dot_general(rowsum, stri, (((0,), (0,)), ((), ())),
                          preferred_element_type=jnp.float32)
    return intra + off[:, None]


def _scores_body(a_ref, bb_ref, idx_ref, sel_ref, ne_ref, no_ref, vals_scr):
    j = pl.program_id(1)
    a = a_ref[0]
    bt = bb_ref[0]
    s = lax.dot_general(a, bt, (((1,), (1,)), ((), ())),
                        preferred_element_type=jnp.float32)
    v = jnp.max(s, axis=1)
    idx_ref[0, 0, 0] = jnp.argmax(s, axis=1).astype(jnp.int32)
    vals_scr[pl.ds(j, 1), :] = v[None, :]

    @pl.when(j == NBLK2 - 1)
    def _():
        vv = vals_scr[...]
        bits = lax.bitcast_convert_type(vv, jnp.int32)
        ikey = jnp.where(bits >= 0, bits, bits ^ jnp.int32(0x7FFFFFFF))
        mi = jnp.int32(MININT)
        tu = jnp.zeros((1, 1), jnp.int32)
        for bit in range(31, -1, -1):
            m = mi if bit == 31 else jnp.int32(1 << bit)
            cand_u = tu | m
            cand_s = cand_u ^ mi
            c = jnp.sum((ikey >= cand_s).astype(jnp.int32), axis=(0, 1),
                        keepdims=True)
            tu = jnp.where(c >= R, cand_u, tu)
        thr = tu ^ mi
        gt = ikey > thr
        eq = ikey == thr
        need = jnp.int32(R) - jnp.sum(gt.astype(jnp.int32), axis=(0, 1),
                                      keepdims=True)
        eqf = eq.astype(jnp.float32)
        r_excl = _cumsum_rows(eqf) - eqf
        sel = gt | (eq & (r_excl < need.astype(jnp.float32)))
        sel32 = sel.astype(jnp.int32)
        s_incl = _cumsum_rows(sel.astype(jnp.float32)).astype(jnp.int32)
        sub = lax.broadcasted_iota(jnp.int32, (NBLK2, BM), 0)
        lane = lax.broadcasted_iota(jnp.int32, (NBLK2, BM), 1)
        k = sub * BM + lane
        sel_ref[0] = sel32
        ne_ref[0] = 2 * k - (s_incl - sel32)
        no_ref[0] = 2 * k + 1 - s_incl


def _scores(a, bb):
    nidx, sel, ne, no = pl.pallas_call(
        _scores_body,
        grid=(B, NBLK2),
        in_specs=[
            pl.BlockSpec((1, BM, MDIM), lambda i, j: (i, j, 0)),
            pl.BlockSpec((1, NH, MDIM), lambda i, j: (i, 0, 0)),
        ],
        out_specs=[
            pl.BlockSpec((1, 1, 1, BM), lambda i, j: (i, j, 0, 0)),
            pl.BlockSpec((1, NBLK2, BM), lambda i, j: (i, 0, 0)),
            pl.BlockSpec((1, NBLK2, BM), lambda i, j: (i, 0, 0)),
            pl.BlockSpec((1, NBLK2, BM), lambda i, j: (i, 0, 0)),
        ],
        out_shape=[
            jax.ShapeDtypeStruct((B, NBLK2, 1, BM), jnp.int32),
            jax.ShapeDtypeStruct((B, NBLK2, BM), jnp.int32),
            jax.ShapeDtypeStruct((B, NBLK2, BM), jnp.int32),
            jax.ShapeDtypeStruct((B, NBLK2, BM), jnp.int32),
        ],
        scratch_shapes=[pltpu.VMEM((NBLK2, BM), jnp.float32)],
    )(a, bb)
    return (nidx.reshape(B, NH), sel.reshape(B, NH), ne.reshape(B, NH),
            no.reshape(B, NH))


def _merge(x, sel, ne, no, nidx):
    mesh = plsc.VectorSubcoreMesh(core_axis_name="c", subcore_axis_name="s",
                                  num_cores=NC, num_subcores=NS)

    def body(x_hbm, sel_hbm, ne_hbm, no_hbm, nidx_hbm, xf_hbm, own_hbm,
             cnt_hbm, sel_t, ne_t, no_t, nidx_t, slots2d, cnt_t, xbuf0,
             xbuf1, zbuf, sem_l, sem_a, sem_d, accum_sp):
        b = lax.axis_index("c")
        t = lax.axis_index("s")
        zero16 = jnp.zeros((16,), jnp.float32)
        ones16 = jnp.ones((16,), jnp.float32)
        iota16 = lax.iota(jnp.int32, 16)

        tl = [pltpu.async_copy(sel_hbm.at[b], sel_t, sem_l),
              pltpu.async_copy(ne_hbm.at[b], ne_t, sem_l),
              pltpu.async_copy(no_hbm.at[b], no_t, sem_l),
              pltpu.async_copy(nidx_hbm.at[b], nidx_t, sem_l)]
        for d in tl:
            d.wait()

        def _zero_cnt(m, carry):
            cnt_t[pl.ds(m * 16, 16)] = zero16
            return carry
        lax.fori_loop(0, NKEEP // 16, _zero_cnt, 0)

        def _zero_zbuf(m, carry):
            zbuf[m // 8, pl.ds((m % 8) * 16, 16)] = zero16
            return carry
        lax.fori_loop(0, 64 * 8, _zero_zbuf, 0)

        def _slots(j, carry):
            i16 = j * 16 + iota16
            selv = sel_t[pl.ds(j * 16, 16)]
            nov = no_t[pl.ds(j * 16, 16)]
            niv = nidx_t[pl.ds(j * 16, 16)]
            nev = ne_t[pl.ds(j * 16, 16)]
            tgt = plsc.load_gather(ne_t, [niv])
            ownodd = jnp.where(selv > 0, tgt, nov)
            pe = i16 * 2
            po = pe + 1
            plsc.store_scatter(slots2d, [pe >> 7, pe & 127], nev)
            plsc.store_scatter(slots2d, [po >> 7, po & 127], ownodd)
            plsc.addupdate_scatter(cnt_t, [nev], ones16)
            plsc.addupdate_scatter(cnt_t, [ownodd], ones16)
            return carry
        lax.fori_loop(0, NH // 16, _slots, 0)

        ow = [pltpu.async_copy(slots2d.at[pl.ds(t * 4, 4)],
                               own_hbm.at[b, pl.ds(t * 4, 4)], sem_d),
              pltpu.async_copy(cnt_t.at[pl.ds(t * QT, QT)],
                               cnt_hbm.at[b, pl.ds(t * QT, QT)], sem_d)]

        def xsrc(g):
            c, u = g // 4, g % 4
            return x_hbm.at[b, pl.ds(t * PT + u * 128, 128),
                            pl.ds(c * CW, CW)]

        nblk = 4 * NCH
        zs = [pltpu.async_copy(zbuf, accum_sp.at[pl.ds(t * QT + z * 64, 64)],
                               sem_a)
              for z in range(QT // 64)]
        lds = [None] * (nblk + 2)
        ads = [None] * nblk
        lds[0] = pltpu.async_copy(xsrc(0), xbuf0, sem_l)
        lds[1] = pltpu.async_copy(xsrc(1), xbuf1, sem_l)
        for d in ow:
            d.wait()
        for d in zs:
            d.wait()
        plsc.subcore_barrier()
        for c in range(NCH):
            for u in range(4):
                g = 4 * c + u
                lds[g].wait()
                buf = xbuf0 if g % 2 == 0 else xbuf1
                ads[g] = pltpu.async_copy(buf,
                                          accum_sp.at[slots2d.at[t * 4 + u]],
                                          sem_a, add=True)
                ads[g].wait()
                if g + 2 < nblk:
                    lds[g + 2] = pltpu.async_copy(xsrc(g + 2), buf, sem_l)
            plsc.subcore_barrier()
            dr = pltpu.async_copy(accum_sp.at[pl.ds(t * QT, QT)],
                                  xf_hbm.at[b, pl.ds(t * QT, QT),
                                            pl.ds(c * CW, CW)], sem_d)
            dr.wait()
            if c + 1 < NCH:
                zs = [pltpu.async_copy(
                          zbuf, accum_sp.at[pl.ds(t * QT + z * 64, 64)],
                          sem_d)
                      for z in range(QT // 64)]
                for d in zs:
                    d.wait()
            plsc.subcore_barrier()

    run = pl.kernel(
        body,
        out_type=(
            jax.ShapeDtypeStruct((B, NKEEP, DIM), jnp.float32),
            jax.ShapeDtypeStruct((B, 64, 128), jnp.int32),
            jax.ShapeDtypeStruct((B, NKEEP), jnp.float32),
        ),
        mesh=mesh,
        compiler_params=pltpu.CompilerParams(needs_layout_passes=False),
        scratch_types=[
            pltpu.VMEM((NH,), jnp.int32),
            pltpu.VMEM((NH,), jnp.int32),
            pltpu.VMEM((NH,), jnp.int32),
            pltpu.VMEM((NH,), jnp.int32),
            pltpu.VMEM((64, 128), jnp.int32),
            pltpu.VMEM((NKEEP,), jnp.float32),
            pltpu.VMEM((128, CW), jnp.float32),
            pltpu.VMEM((128, CW), jnp.float32),
            pltpu.VMEM((64, 128), jnp.float32),
            pltpu.SemaphoreType.DMA,
            pltpu.SemaphoreType.DMA,
            pltpu.SemaphoreType.DMA,
            pltpu.VMEM_SHARED((NKEEP, CW), jnp.float32),
        ],
    )
    xf_raw, own, cnt = run(x, sel, ne, no, nidx)
    return xf_raw, own.reshape(B, N), cnt


def _divide_body(xr_ref, c_ref, out_ref):
    out_ref[0] = xr_ref[0] / c_ref[0]


def _divide(xf_raw, cnt):
    blk = 512
    return pl.pallas_call(
        _divide_body,
        grid=(B, NKEEP // blk),
        in_specs=[
            pl.BlockSpec((1, blk, DIM), lambda i, j: (i, j, 0)),
            pl.BlockSpec((1, blk, 1), lambda i, j: (i, j, 0)),
        ],
        out_specs=pl.BlockSpec((1, blk, DIM), lambda i, j: (i, j, 0)),
        out_shape=jax.ShapeDtypeStruct((B, NKEEP, DIM), jnp.float32),
    )(xf_raw, cnt.reshape(B, NKEEP, 1))


def kernel(x, W, b):
    a, bb = _mnorm(x, W, b)
    nidx, sel, ne, no = _scores(a, bb)
    xf_raw, own, cnt = _merge(x, sel, ne, no, nidx)
    xf = _divide(xf_raw, cnt)
    return (xf, own)


# speedup vs baseline: 2.4681x; 1.0159x over previous
import functools

import jax
import jax.numpy as jnp
from jax import lax
from jax.experimental import pallas as pl
from jax.experimental.pallas import tpu as pltpu
from jax.experimental.pallas import tpu_sc as plsc

DIM = 768
MDIM = DIM // 4
N = 8192
NH = N // 2
R = 2048
NKEEP = N - R
B = 2

BLK1 = 512
BM = 512
CW = 128
NCH = DIM // CW
NC, NS = 2, 16
PT = N // NS
QT = NKEEP // NS

MININT = -(2 ** 31)


NBLK2 = NH // BM


def _cumsum_rows(x):
    li = lax.broadcasted_iota(jnp.int32, (BM, BM), 0)
    lj = lax.broadcasted_iota(jnp.int32, (BM, BM), 1)
    tri = (li <= lj).astype(jnp.float32)
    intra = lax.dot_general(x, tri, (((1,), (0,)), ((), ())),
                            preferred_element_type=jnp.float32)
    rowsum = jnp.sum(x, axis=1)
    si = lax.broadcasted_iota(jnp.int32, (NBLK2, NBLK2), 0)
    sj = lax.broadcasted_iota(jnp.int32, (NBLK2, NBLK2), 1)
    stri = (si < sj).astype(jnp.float32)
    off = lax.dot_general(rowsum, stri, (((0,), (0,)), ((), ())),
                          preferred_element_type=jnp.float32)
    return intra + off[:, None]


NB1 = NH // BLK1


def _scores_body(x_ref, w_ref, b_ref, idx_ref, sel_ref, ne_ref, no_ref,
                 a_scr, bb_scr, vals_scr):
    j = pl.program_id(1)

    @pl.when(j < NB1)
    def _():
        xab = x_ref[0]
        w = w_ref[...]
        bv = b_ref[...]
        ma = lax.dot_general(xab[:, :DIM], w, (((1,), (1,)), ((), ())),
                             preferred_element_type=jnp.float32) + bv
        mb = lax.dot_general(xab[:, DIM:], w, (((1,), (1,)), ((), ())),
                             preferred_element_type=jnp.float32) + bv
        sl = pl.ds(j * BLK1, BLK1)
        a_scr[sl, :] = ma / jnp.sqrt(jnp.sum(ma * ma, axis=1, keepdims=True))
        bb_scr[sl, :] = mb / jnp.sqrt(jnp.sum(mb * mb, axis=1, keepdims=True))

    @pl.when(j >= NB1)
    def _():
        jb = j - NB1
        a = a_scr[pl.ds(jb * BM, BM), :]
        bt = bb_scr[...]
        s = lax.dot_general(a, bt, (((1,), (1,)), ((), ())),
                            preferred_element_type=jnp.float32)
        v = jnp.max(s, axis=1)
        idx_ref[0, 0, 0] = jnp.argmax(s, axis=1).astype(jnp.int32)
        vals_scr[pl.ds(jb, 1), :] = v[None, :]

    @pl.when(j == NB1 + NBLK2 - 1)
    def _():
        vv = vals_scr[...]
        bits = lax.bitcast_convert_type(vv, jnp.int32)
        ikey = jnp.where(bits >= 0, bits, bits ^ jnp.int32(0x7FFFFFFF))
        mi = jnp.int32(MININT)
        tu = jnp.zeros((1, 1), jnp.int32)
        for bit in range(31, -1, -1):
            m = mi if bit == 31 else jnp.int32(1 << bit)
            cand_u = tu | m
            cand_s = cand_u ^ mi
            c = jnp.sum((ikey >= cand_s).astype(jnp.int32), axis=(0, 1),
                        keepdims=True)
            tu = jnp.where(c >= R, cand_u, tu)
        thr = tu ^ mi
        gt = ikey > thr
        eq = ikey == thr
        need = jnp.int32(R) - jnp.sum(gt.astype(jnp.int32), axis=(0, 1),
                                      keepdims=True)
        eqf = eq.astype(jnp.float32)
        r_excl = _cumsum_rows(eqf) - eqf
        sel = gt | (eq & (r_excl < need.astype(jnp.float32)))
        sel32 = sel.astype(jnp.int32)
        s_incl = _cumsum_rows(sel.astype(jnp.float32)).astype(jnp.int32)
        sub = lax.broadcasted_iota(jnp.int32, (NBLK2, BM), 0)
        lane = lax.broadcasted_iota(jnp.int32, (NBLK2, BM), 1)
        k = sub * BM + lane
        sel_ref[0] = sel32
        ne_ref[0] = 2 * k - (s_incl - sel32)
        no_ref[0] = 2 * k + 1 - s_incl


def _scores(x, W, bvec):
    xr = x.reshape(B, NH, 2 * DIM)
    nidx, sel, ne, no = pl.pallas_call(
        _scores_body,
        grid=(B, NB1 + NBLK2),
        in_specs=[
            pl.BlockSpec((1, BLK1, 2 * DIM),
                         lambda i, j: (i, jnp.where(j < NB1, j, 0), 0)),
            pl.BlockSpec((MDIM, DIM), lambda i, j: (0, 0)),
            pl.BlockSpec((1, MDIM), lambda i, j: (0, 0)),
        ],
        out_specs=[
            pl.BlockSpec((1, 1, 1, BM),
                         lambda i, j: (i, jnp.where(j >= NB1, j - NB1, 0),
                                       0, 0)),
            pl.BlockSpec((1, NBLK2, BM), lambda i, j: (i, 0, 0)),
            pl.BlockSpec((1, NBLK2, BM), lambda i, j: (i, 0, 0)),
            pl.BlockSpec((1, NBLK2, BM), lambda i, j: (i, 0, 0)),
        ],
        out_shape=[
            jax.ShapeDtypeStruct((B, NBLK2, 1, BM), jnp.int32),
            jax.ShapeDtypeStruct((B, NBLK2, BM), jnp.int32),
            jax.ShapeDtypeStruct((B, NBLK2, BM), jnp.int32),
            jax.ShapeDtypeStruct((B, NBLK2, BM), jnp.int32),
        ],
        scratch_shapes=[
            pltpu.VMEM((NH, MDIM), jnp.float32),
            pltpu.VMEM((NH, MDIM), jnp.float32),
            pltpu.VMEM((NBLK2, BM), jnp.float32),
        ],
    )(xr, W, bvec.reshape(1, MDIM))
    return (nidx.reshape(B, NH), sel.reshape(B, NH), ne.reshape(B, NH),
            no.reshape(B, NH))


def _merge(x, sel, ne, no, nidx):
    mesh = plsc.VectorSubcoreMesh(core_axis_name="c", subcore_axis_name="s",
                                  num_cores=NC, num_subcores=NS)

    def body(x_hbm, sel_hbm, ne_hbm, no_hbm, nidx_hbm, xf_hbm, own_hbm,
             cnt_hbm, sel_t, ne_t, no_t, nidx_t, slots2d, cnt_t, xbuf0,
             xbuf1, zbuf, sem_l, sem_a, sem_d, accum_sp):
        b = lax.axis_index("c")
        t = lax.axis_index("s")
        zero16 = jnp.zeros((16,), jnp.float32)
        ones16 = jnp.ones((16,), jnp.float32)
        iota16 = lax.iota(jnp.int32, 16)

        tl = [pltpu.async_copy(sel_hbm.at[b], sel_t, sem_l),
              pltpu.async_copy(ne_hbm.at[b], ne_t, sem_l),
              pltpu.async_copy(no_hbm.at[b], no_t, sem_l),
              pltpu.async_copy(nidx_hbm.at[b], nidx_t, sem_l)]
        for d in tl:
            d.wait()

        def _zero_cnt(m, carry):
            cnt_t[pl.ds(m * 16, 16)] = zero16
            return carry
        lax.fori_loop(0, NKEEP // 16, _zero_cnt, 0)

        def _zero_zbuf(m, carry):
            zbuf[m // 8, pl.ds((m % 8) * 16, 16)] = zero16
            return carry
        lax.fori_loop(0, 64 * 8, _zero_zbuf, 0)

        def _slots(j, carry):
            i16 = j * 16 + iota16
            selv = sel_t[pl.ds(j * 16, 16)]
            nov = no_t[pl.ds(j * 16, 16)]
            niv = nidx_t[pl.ds(j * 16, 16)]
            nev = ne_t[pl.ds(j * 16, 16)]
            tgt = plsc.load_gather(ne_t, [niv])
            ownodd = jnp.where(selv > 0, tgt, nov)
            pe = i16 * 2
            po = pe + 1
            plsc.store_scatter(slots2d, [pe >> 7, pe & 127], nev)
            plsc.store_scatter(slots2d, [po >> 7, po & 127], ownodd)
            plsc.addupdate_scatter(cnt_t, [nev], ones16)
            plsc.addupdate_scatter(cnt_t, [ownodd], ones16)
            return carry
        lax.fori_loop(0, NH // 16, _slots, 0)

        ow = [pltpu.async_copy(slots2d.at[pl.ds(t * 4, 4)],
                               own_hbm.at[b, pl.ds(t * 4, 4)], sem_d),
              pltpu.async_copy(cnt_t.at[pl.ds(t * QT, QT)],
                               cnt_hbm.at[b, pl.ds(t * QT, QT)], sem_d)]

        def xsrc(g):
            c, u = g // 4, g % 4
            return x_hbm.at[b, pl.ds(t * PT + u * 128, 128),
                            pl.ds(c * CW, CW)]

        nblk = 4 * NCH
        zs = [pltpu.async_copy(zbuf, accum_sp.at[pl.ds(t * QT + z * 64, 64)],
                               sem_a)
              for z in range(QT // 64)]
        lds = [None] * (nblk + 2)
        ads = [None] * nblk
        lds[0] = pltpu.async_copy(xsrc(0), xbuf0, sem_l)
        lds[1] = pltpu.async_copy(xsrc(1), xbuf1, sem_l)
        for d in ow:
            d.wait()
        for d in zs:
            d.wait()
        plsc.subcore_barrier()
        for c in range(NCH):
            for u in range(4):
                g = 4 * c + u
                lds[g].wait()
                buf = xbuf0 if g % 2 == 0 else xbuf1
                ads[g] = pltpu.async_copy(buf,
                                          accum_sp.at[slots2d.at[t * 4 + u]],
                                          sem_a, add=True)
                ads[g].wait()
                if g + 2 < nblk:
                    lds[g + 2] = pltpu.async_copy(xsrc(g + 2), buf, sem_l)
            plsc.subcore_barrier()
            dr = pltpu.async_copy(accum_sp.at[pl.ds(t * QT, QT)],
                                  xf_hbm.at[b, pl.ds(t * QT, QT),
                                            pl.ds(c * CW, CW)], sem_d)
            dr.wait()
            if c + 1 < NCH:
                zs = [pltpu.async_copy(
                          zbuf, accum_sp.at[pl.ds(t * QT + z * 64, 64)],
                          sem_d)
                      for z in range(QT // 64)]
                for d in zs:
                    d.wait()
            plsc.subcore_barrier()

    run = pl.kernel(
        body,
        out_type=(
            jax.ShapeDtypeStruct((B, NKEEP, DIM), jnp.float32),
            jax.ShapeDtypeStruct((B, 64, 128), jnp.int32),
            jax.ShapeDtypeStruct((B, NKEEP), jnp.float32),
        ),
        mesh=mesh,
        compiler_params=pltpu.CompilerParams(needs_layout_passes=False),
        scratch_types=[
            pltpu.VMEM((NH,), jnp.int32),
            pltpu.VMEM((NH,), jnp.int32),
            pltpu.VMEM((NH,), jnp.int32),
            pltpu.VMEM((NH,), jnp.int32),
            pltpu.VMEM((64, 128), jnp.int32),
            pltpu.VMEM((NKEEP,), jnp.float32),
            pltpu.VMEM((128, CW), jnp.float32),
            pltpu.VMEM((128, CW), jnp.float32),
            pltpu.VMEM((64, 128), jnp.float32),
            pltpu.SemaphoreType.DMA,
            pltpu.SemaphoreType.DMA,
            pltpu.SemaphoreType.DMA,
            pltpu.VMEM_SHARED((NKEEP, CW), jnp.float32),
        ],
    )
    xf_raw, own, cnt = run(x, sel, ne, no, nidx)
    return xf_raw, own.reshape(B, N), cnt


def _divide_body(xr_ref, c_ref, out_ref):
    out_ref[0] = xr_ref[0] / c_ref[0]


def _divide(xf_raw, cnt):
    blk = 512
    return pl.pallas_call(
        _divide_body,
        grid=(B, NKEEP // blk),
        in_specs=[
            pl.BlockSpec((1, blk, DIM), lambda i, j: (i, j, 0)),
            pl.BlockSpec((1, blk, 1), lambda i, j: (i, j, 0)),
        ],
        out_specs=pl.BlockSpec((1, blk, DIM), lambda i, j: (i, j, 0)),
        out_shape=jax.ShapeDtypeStruct((B, NKEEP, DIM), jnp.float32),
    )(xf_raw, cnt.reshape(B, NKEEP, 1))


def kernel(x, W, b):
    nidx, sel, ne, no = _scores(x, W, b)
    xf_raw, own, cnt = _merge(x, sel, ne, no, nidx)
    xf = _divide(xf_raw, cnt)
    return (xf, own)


# speedup vs baseline: 2.4736x; 1.0022x over previous
import jax
import jax.numpy as jnp
from jax import lax
from jax.experimental import pallas as pl
from jax.experimental.pallas import tpu as pltpu
from jax.experimental.pallas import tpu_sc as plsc

DIM = 768
MDIM = DIM // 4
N = 8192
NH = N // 2
R = 2048
NKEEP = N - R
B = 2

BLK1 = 512
BM = 512
CW = 128
NCH = DIM // CW
NC, NS = 2, 16
PT = N // NS
QT = NKEEP // NS

MININT = -(2 ** 31)


NBLK2 = NH // BM


def _cumsum_rows(x):
    li = lax.broadcasted_iota(jnp.int32, (BM, BM), 0)
    lj = lax.broadcasted_iota(jnp.int32, (BM, BM), 1)
    tri = (li <= lj).astype(jnp.float32)
    intra = lax.dot_general(x, tri, (((1,), (0,)), ((), ())),
                            preferred_element_type=jnp.float32)
    rowsum = jnp.sum(x, axis=1)
    si = lax.broadcasted_iota(jnp.int32, (NBLK2, NBLK2), 0)
    sj = lax.broadcasted_iota(jnp.int32, (NBLK2, NBLK2), 1)
    stri = (si < sj).astype(jnp.float32)
    off = lax.dot_general(rowsum, stri, (((0,), (0,)), ((), ())),
                          preferred_element_type=jnp.float32)
    return intra + off[:, None]


NB1 = NH // BLK1


def _scores_body(x_ref, w_ref, b_ref, idx_ref, sel_ref, ne_ref, no_ref,
                 a_scr, bb_scr, vals_scr):
    j = pl.program_id(1)

    @pl.when(j < NB1)
    def _():
        xab = x_ref[0]
        w = w_ref[...]
        bv = b_ref[...]
        ma = lax.dot_general(xab[:, :DIM], w, (((1,), (1,)), ((), ())),
                             preferred_element_type=jnp.float32) + bv
        mb = lax.dot_general(xab[:, DIM:], w, (((1,), (1,)), ((), ())),
                             preferred_element_type=jnp.float32) + bv
        sl = pl.ds(j * BLK1, BLK1)
        a_scr[sl, :] = ma / jnp.sqrt(jnp.sum(ma * ma, axis=1, keepdims=True))
        bb_scr[sl, :] = mb / jnp.sqrt(jnp.sum(mb * mb, axis=1, keepdims=True))

    @pl.when(j >= NB1)
    def _():
        jb = j - NB1
        a = a_scr[pl.ds(jb * BM, BM), :]
        bt = bb_scr[...]
        s = lax.dot_general(a, bt, (((1,), (1,)), ((), ())),
                            preferred_element_type=jnp.float32)
        v = jnp.max(s, axis=1)
        idx_ref[0, 0, 0] = jnp.argmax(s, axis=1).astype(jnp.int32)
        vals_scr[pl.ds(jb, 1), :] = v[None, :]

    @pl.when(j == NB1 + NBLK2 - 1)
    def _():
        vv = vals_scr[...]
        bits = lax.bitcast_convert_type(vv, jnp.int32)
        ikey = jnp.where(bits >= 0, bits, bits ^ jnp.int32(0x7FFFFFFF))
        mi = jnp.int32(MININT)
        tu = jnp.zeros((1, 1), jnp.int32)
        for bit in range(31, -1, -1):
            m = mi if bit == 31 else jnp.int32(1 << bit)
            cand_u = tu | m
            cand_s = cand_u ^ mi
            c = jnp.sum((ikey >= cand_s).astype(jnp.int32), axis=(0, 1),
                        keepdims=True)
            tu = jnp.where(c >= R, cand_u, tu)
        thr = tu ^ mi
        gt = ikey > thr
        eq = ikey == thr
        need = jnp.int32(R) - jnp.sum(gt.astype(jnp.int32), axis=(0, 1),
                                      keepdims=True)
        eqf = eq.astype(jnp.float32)
        r_excl = _cumsum_rows(eqf) - eqf
        sel = gt | (eq & (r_excl < need.astype(jnp.float32)))
        sel32 = sel.astype(jnp.int32)
        s_incl = _cumsum_rows(sel.astype(jnp.float32)).astype(jnp.int32)
        sub = lax.broadcasted_iota(jnp.int32, (NBLK2, BM), 0)
        lane = lax.broadcasted_iota(jnp.int32, (NBLK2, BM), 1)
        k = sub * BM + lane
        sel_ref[0] = sel32
        ne_ref[0] = 2 * k - (s_incl - sel32)
        no_ref[0] = 2 * k + 1 - s_incl


def _scores(x, W, bvec):
    xr = x.reshape(B, NH, 2 * DIM)
    nidx, sel, ne, no = pl.pallas_call(
        _scores_body,
        grid=(B, NB1 + NBLK2),
        in_specs=[
            pl.BlockSpec((1, BLK1, 2 * DIM),
                         lambda i, j: (i, jnp.where(j < NB1, j, 0), 0)),
            pl.BlockSpec((MDIM, DIM), lambda i, j: (0, 0)),
            pl.BlockSpec((1, MDIM), lambda i, j: (0, 0)),
        ],
        out_specs=[
            pl.BlockSpec((1, 1, 1, BM),
                         lambda i, j: (i, jnp.where(j >= NB1, j - NB1, 0),
                                       0, 0)),
            pl.BlockSpec((1, NBLK2, BM), lambda i, j: (i, 0, 0)),
            pl.BlockSpec((1, NBLK2, BM), lambda i, j: (i, 0, 0)),
            pl.BlockSpec((1, NBLK2, BM), lambda i, j: (i, 0, 0)),
        ],
        out_shape=[
            jax.ShapeDtypeStruct((B, NBLK2, 1, BM), jnp.int32),
            jax.ShapeDtypeStruct((B, NBLK2, BM), jnp.int32),
            jax.ShapeDtypeStruct((B, NBLK2, BM), jnp.int32),
            jax.ShapeDtypeStruct((B, NBLK2, BM), jnp.int32),
        ],
        scratch_shapes=[
            pltpu.VMEM((NH, MDIM), jnp.float32),
            pltpu.VMEM((NH, MDIM), jnp.float32),
            pltpu.VMEM((NBLK2, BM), jnp.float32),
        ],
    )(xr, W, bvec.reshape(1, MDIM))
    return (nidx.reshape(B, NH), sel.reshape(B, NH), ne.reshape(B, NH),
            no.reshape(B, NH))


def _merge(x, sel, ne, no, nidx):
    mesh = plsc.VectorSubcoreMesh(core_axis_name="c", subcore_axis_name="s",
                                  num_cores=NC, num_subcores=NS)

    def body(x_hbm, sel_hbm, ne_hbm, no_hbm, nidx_hbm, xf_hbm, own_hbm,
             cnt_hbm, sel_t, ne_t, no_t, nidx_t, slots2d, cnt_t, xbuf0,
             xbuf1, zbuf, sem_l, sem_a, sem_d, accum_sp):
        b = lax.axis_index("c")
        t = lax.axis_index("s")
        zero16 = jnp.zeros((16,), jnp.float32)
        ones16 = jnp.ones((16,), jnp.float32)
        iota16 = lax.iota(jnp.int32, 16)

        tl = [pltpu.async_copy(sel_hbm.at[b], sel_t, sem_l),
              pltpu.async_copy(ne_hbm.at[b], ne_t, sem_l),
              pltpu.async_copy(no_hbm.at[b], no_t, sem_l),
              pltpu.async_copy(nidx_hbm.at[b], nidx_t, sem_l)]
        for d in tl:
            d.wait()

        def _zero_cnt(m, carry):
            cnt_t[pl.ds(m * 16, 16)] = zero16
            return carry
        lax.fori_loop(0, NKEEP // 16, _zero_cnt, 0)

        def _zero_zbuf(m, carry):
            zbuf[m // 8, pl.ds((m % 8) * 16, 16)] = zero16
            return carry
        lax.fori_loop(0, 64 * 8, _zero_zbuf, 0)

        def _slots(j, carry):
            i16 = j * 16 + iota16
            selv = sel_t[pl.ds(j * 16, 16)]
            nov = no_t[pl.ds(j * 16, 16)]
            niv = nidx_t[pl.ds(j * 16, 16)]
            nev = ne_t[pl.ds(j * 16, 16)]
            tgt = plsc.load_gather(ne_t, [niv])
            ownodd = jnp.where(selv > 0, tgt, nov)
            pe = i16 * 2
            po = pe + 1
            plsc.store_scatter(slots2d, [pe >> 7, pe & 127], nev)
            plsc.store_scatter(slots2d, [po >> 7, po & 127], ownodd)
            plsc.addupdate_scatter(cnt_t, [nev], ones16)
            plsc.addupdate_scatter(cnt_t, [ownodd], ones16)
            return carry
        lax.fori_loop(0, NH // 16, _slots, 0)

        ow = [pltpu.async_copy(slots2d.at[pl.ds(t * 4, 4)],
                               own_hbm.at[b, pl.ds(t * 4, 4)], sem_d),
              pltpu.async_copy(cnt_t.at[pl.ds(t * QT, QT)],
                               cnt_hbm.at[b, pl.ds(t * QT, QT)], sem_d)]

        def xsrc(g):
            c, u = g // 4, g % 4
            return x_hbm.at[b, pl.ds(t * PT + u * 128, 128),
                            pl.ds(c * CW, CW)]

        nblk = 4 * NCH
        zs = [pltpu.async_copy(zbuf, accum_sp.at[pl.ds(t * QT + z * 64, 64)],
                               sem_a)
              for z in range(QT // 64)]
        lds = [None] * (nblk + 2)
        ads = [None] * nblk
        lds[0] = pltpu.async_copy(xsrc(0), xbuf0, sem_l)
        lds[1] = pltpu.async_copy(xsrc(1), xbuf1, sem_l)
        for d in ow:
            d.wait()
        for d in zs:
            d.wait()
        plsc.subcore_barrier()
        for c in range(NCH):
            for u in range(4):
                g = 4 * c + u
                lds[g].wait()
                buf = xbuf0 if g % 2 == 0 else xbuf1
                ads[g] = pltpu.async_copy(buf,
                                          accum_sp.at[slots2d.at[t * 4 + u]],
                                          sem_a, add=True)
                ads[g].wait()
                if g + 2 < nblk:
                    lds[g + 2] = pltpu.async_copy(xsrc(g + 2), buf, sem_l)
            plsc.subcore_barrier()
            dr = pltpu.async_copy(accum_sp.at[pl.ds(t * QT, QT)],
                                  xf_hbm.at[b, pl.ds(t * QT, QT),
                                            pl.ds(c * CW, CW)], sem_d)
            dr.wait()
            if c + 1 < NCH:
                zs = [pltpu.async_copy(
                          zbuf, accum_sp.at[pl.ds(t * QT + z * 64, 64)],
                          sem_d)
                      for z in range(QT // 64)]
                for d in zs:
                    d.wait()
            plsc.subcore_barrier()

    run = pl.kernel(
        body,
        out_type=(
            jax.ShapeDtypeStruct((B, NKEEP, DIM), jnp.float32),
            jax.ShapeDtypeStruct((B, 64, 128), jnp.int32),
            jax.ShapeDtypeStruct((B, NKEEP), jnp.float32),
        ),
        mesh=mesh,
        compiler_params=pltpu.CompilerParams(needs_layout_passes=False),
        scratch_types=[
            pltpu.VMEM((NH,), jnp.int32),
            pltpu.VMEM((NH,), jnp.int32),
            pltpu.VMEM((NH,), jnp.int32),
            pltpu.VMEM((NH,), jnp.int32),
            pltpu.VMEM((64, 128), jnp.int32),
            pltpu.VMEM((NKEEP,), jnp.float32),
            pltpu.VMEM((128, CW), jnp.float32),
            pltpu.VMEM((128, CW), jnp.float32),
            pltpu.VMEM((64, 128), jnp.float32),
            pltpu.SemaphoreType.DMA,
            pltpu.SemaphoreType.DMA,
            pltpu.SemaphoreType.DMA,
            pltpu.VMEM_SHARED((NKEEP, CW), jnp.float32),
        ],
    )
    xf_raw, own, cnt = run(x, sel, ne, no, nidx)
    return xf_raw, own.reshape(B, N), cnt


def _divide_body(xr_ref, c_ref, out_ref):
    out_ref[0] = xr_ref[0] / c_ref[0]


def _divide(xf_raw, cnt):
    blk = 512
    return pl.pallas_call(
        _divide_body,
        grid=(B, NKEEP // blk),
        in_specs=[
            pl.BlockSpec((1, blk, DIM), lambda i, j: (i, j, 0)),
            pl.BlockSpec((1, blk, 1), lambda i, j: (i, j, 0)),
        ],
        out_specs=pl.BlockSpec((1, blk, DIM), lambda i, j: (i, j, 0)),
        out_shape=jax.ShapeDtypeStruct((B, NKEEP, DIM), jnp.float32),
    )(xf_raw, cnt.reshape(B, NKEEP, 1))


def kernel(x, W, b):
    nidx, sel, ne, no = _scores(x, W, b)
    xf_raw, own, cnt = _merge(x, sel, ne, no, nidx)
    xf = _divide(xf_raw, cnt)
    return (xf, own)
